# baseline (speedup 1.0000x reference)
import functools

import jax
import jax.numpy as jnp
from jax import lax
from jax.experimental import pallas as pl
from jax.experimental.pallas import tpu as pltpu
from jax.experimental.pallas import tpu_sc as plsc

N_NODES = 10000
N_EDGES = 320000
F = 128
FH = F // 2

NC = 2
NS = 16
EB = 128
N_BATCH = 160
EDGES_PER_TILE = N_BATCH * EB
E_PAD = NS * EDGES_PER_TILE
DEAD_ROW = 10200
ACC_ROWS = 10240
ROWS_PER_TILE = ACC_ROWS // NS
ZR = 8
NBUF = 4


def _sc_scatter(x2, row3, col4):
    mesh = plsc.VectorSubcoreMesh(
        core_axis_name="c", subcore_axis_name="s",
        num_cores=NC, num_subcores=NS)

    @functools.partial(
        pl.kernel,
        out_type=jax.ShapeDtypeStruct((NC, ACC_ROWS, FH), jnp.float32),
        mesh=mesh,
        scratch_types=[
            pltpu.VMEM((N_BATCH, EB), jnp.int32),
            pltpu.VMEM((N_BATCH, EB), jnp.int32),
            [pltpu.VMEM((EB, FH), jnp.float32) for _ in range(NBUF)],
            pltpu.VMEM((ZR, FH), jnp.float32),
            pltpu.VMEM_SHARED((ACC_ROWS, FH), jnp.float32),
            [pltpu.SemaphoreType.DMA for _ in range(NBUF)],
            pltpu.SemaphoreType.DMA,
            pltpu.SemaphoreType.DMA,
        ],
        compiler_params=pltpu.CompilerParams(use_tc_tiling_on_sc=False),
    )
    def k(x2_hbm, row3_hbm, col4_hbm, out_hbm, cidx_v, ridx_v, bufs,
          zbuf_v, acc_sh, sems, sem_z, sem_i):
        c = lax.axis_index("c")
        s = lax.axis_index("s")

        idx_cp_c = pltpu.async_copy(col4_hbm.at[c].at[s], cidx_v, sem_i)
        idx_cp_r = pltpu.async_copy(row3_hbm.at[s], ridx_v, sem_z)

        zero16 = jnp.zeros((16,), jnp.float32)
        for i in range(ZR):
            for j in range(FH // 16):
                zbuf_v[i, pl.ds(j * 16, 16)] = zero16

        base_row = s * ROWS_PER_TILE
        idx_cp_c.wait()
        idx_cp_r.wait()

        def zfire(i, carry):
            pltpu.async_copy(zbuf_v, acc_sh.at[pl.ds(base_row + i * ZR, ZR)],
                             sem_z)
            return carry

        def zdrain(i, carry):
            pltpu.make_async_copy(
                zbuf_v, acc_sh.at[pl.ds(base_row + i * ZR, ZR)],
                sem_z).wait()
            return carry

        lax.fori_loop(0, ROWS_PER_TILE // ZR, zfire, 0)
        lax.fori_loop(0, ROWS_PER_TILE // ZR, zdrain, 0)
        plsc.subcore_barrier()

        def gather(i, b):
            pltpu.async_copy(x2_hbm.at[cidx_v.at[i]], bufs[b], sems[b])

        def gwait(i, b):
            pltpu.make_async_copy(
                x2_hbm.at[cidx_v.at[i]], bufs[b], sems[b]).wait()

        def scat(i, b):
            pltpu.sync_copy(bufs[b], acc_sh.at[ridx_v.at[i]], add=True)

        for b in range(NBUF):
            gather(b, b)

        def eloop(j, carry):
            i = j * NBUF
            for b in range(NBUF):
                gwait(i + b, b)
                scat(i + b, b)
                gather(i + NBUF + b, b)
            return carry

        lax.fori_loop(0, N_BATCH // NBUF - 1, eloop, 0)
        for b in range(NBUF):
            i = N_BATCH - NBUF + b
            gwait(i, b)
            scat(i, b)
        plsc.subcore_barrier()

        pltpu.sync_copy(acc_sh.at[pl.ds(base_row, ROWS_PER_TILE)],
                        out_hbm.at[c].at[pl.ds(base_row, ROWS_PER_TILE)])

    return k(x2, row3, col4)


def _tc_combine(partials, W, b2d):
    BR = 2560

    def body(p_ref, w_ref, b_ref, o_ref):
        o_ref[...] = (
            jnp.dot(p_ref[0], w_ref[:FH, :],
                    preferred_element_type=jnp.float32)
            + jnp.dot(p_ref[1], w_ref[FH:, :],
                      preferred_element_type=jnp.float32)
            + b_ref[...])

    return pl.pallas_call(
        body,
        grid=(ACC_ROWS // BR,),
        in_specs=[
            pl.BlockSpec((NC, BR, FH), lambda i: (0, i, 0)),
            pl.BlockSpec((F, F), lambda i: (0, 0)),
            pl.BlockSpec((1, F), lambda i: (0, 0)),
        ],
        out_specs=pl.BlockSpec((BR, F), lambda i: (i, 0)),
        out_shape=jax.ShapeDtypeStruct((N_NODES, F), jnp.float32),
    )(partials, W, b2d)


def kernel(x, edge_index_or_adj, W, b):
    ei = edge_index_or_adj.astype(jnp.int32)
    pad = E_PAD - N_EDGES
    rowp = jnp.concatenate(
        [ei[0], jnp.full((pad,), DEAD_ROW, jnp.int32)])
    colp = jnp.concatenate([ei[1], jnp.zeros((pad,), jnp.int32)])
    row3 = rowp.reshape(NS, N_BATCH, EB)
    col3 = colp.reshape(NS, N_BATCH, EB)
    col4 = jnp.stack([col3, col3 + N_NODES], axis=0)
    x2 = x.reshape(N_NODES, NC, FH).swapaxes(0, 1).reshape(NC * N_NODES, FH)
    partials = _sc_scatter(x2, row3, col4)
    return _tc_combine(partials, W, b.reshape(1, F))


# speedup vs baseline: 2.2628x; 2.2628x over previous
import functools

import jax
import jax.numpy as jnp
from jax import lax
from jax.experimental import pallas as pl
from jax.experimental.pallas import tpu as pltpu
from jax.experimental.pallas import tpu_sc as plsc

N_NODES = 10000
N_EDGES = 320000
F = 128

NC = 2
NS = 16
EDGES_PER_CORE = N_EDGES // NC
EDGES_PER_TILE = EDGES_PER_CORE // NS
EB = 80
N_BATCH = EDGES_PER_TILE // EB
ACC_ROWS = 10240
ROWS_PER_TILE = ACC_ROWS // NS
ZR = 8


def _sc_scatter(x, row, col):
    mesh = plsc.VectorSubcoreMesh(
        core_axis_name="c", subcore_axis_name="s",
        num_cores=NC, num_subcores=NS)

    @functools.partial(
        pl.kernel,
        out_type=jax.ShapeDtypeStruct((NC, ACC_ROWS, F), jnp.float32),
        mesh=mesh,
        scratch_types=[
            pltpu.VMEM((N_BATCH * EB,), jnp.int32),
            pltpu.VMEM((N_BATCH, EB), jnp.int32),
            pltpu.VMEM((EB, F), jnp.float32),
            pltpu.VMEM((EB, F), jnp.float32),
            pltpu.VMEM((ZR, F), jnp.float32),
            pltpu.VMEM_SHARED((ACC_ROWS, F), jnp.float32),
            pltpu.SemaphoreType.DMA,
            pltpu.SemaphoreType.DMA,
            pltpu.SemaphoreType.DMA,
        ],
    )
    def k(x_hbm, row2d_hbm, col_hbm, out_hbm, cidx_v, ridx_v, buf_a, buf_b,
          zbuf_v, acc_sh, sem_a, sem_b, sem_z):
        c = lax.axis_index("c")
        s = lax.axis_index("s")

        tid = c * NS + s
        idx_cp_c = pltpu.async_copy(
            col_hbm.at[pl.ds(tid * EDGES_PER_TILE, EDGES_PER_TILE)],
            cidx_v, sem_a)
        idx_cp_r = pltpu.async_copy(row2d_hbm.at[tid], ridx_v, sem_b)

        zero16 = jnp.zeros((16,), jnp.float32)
        for i in range(ZR):
            for j in range(F // 16):
                zbuf_v[i, pl.ds(j * 16, 16)] = zero16

        base_row = s * ROWS_PER_TILE

        def zfire(i, carry):
            pltpu.async_copy(zbuf_v, acc_sh.at[pl.ds(base_row + i * ZR, ZR)],
                             sem_z)
            return carry

        def zdrain(i, carry):
            pltpu.make_async_copy(
                zbuf_v, acc_sh.at[pl.ds(base_row + i * ZR, ZR)],
                sem_z).wait()
            return carry

        def gather(i, buf, sem):
            pltpu.async_copy(x_hbm.at[cidx_v.at[pl.ds(i * EB, EB)]], buf, sem)

        def gwait(i, buf, sem):
            pltpu.make_async_copy(
                x_hbm.at[cidx_v.at[pl.ds(i * EB, EB)]], buf, sem).wait()

        def scat(i, buf):
            pltpu.sync_copy(buf, acc_sh.at[ridx_v.at[i]], add=True)

        lax.fori_loop(0, ROWS_PER_TILE // ZR, zfire, 0)
        idx_cp_c.wait()
        idx_cp_r.wait()
        gather(0, buf_a, sem_a)
        gather(1, buf_b, sem_b)
        lax.fori_loop(0, ROWS_PER_TILE // ZR, zdrain, 0)
        plsc.subcore_barrier()

        def eloop(j, carry):
            i = 2 * j
            gwait(i, buf_a, sem_a)
            scat(i, buf_a)
            gather(i + 2, buf_a, sem_a)
            gwait(i + 1, buf_b, sem_b)
            scat(i + 1, buf_b)
            gather(i + 3, buf_b, sem_b)
            return carry

        n_even = (N_BATCH // 2) * 2
        lax.fori_loop(0, n_even // 2 - 1, eloop, 0)
        gwait(n_even - 2, buf_a, sem_a)
        scat(n_even - 2, buf_a)
        gwait(n_even - 1, buf_b, sem_b)
        scat(n_even - 1, buf_b)
        for i in range(n_even, N_BATCH):
            gather(i, buf_a, sem_a)
            gwait(i, buf_a, sem_a)
            scat(i, buf_a)
        plsc.subcore_barrier()

        pltpu.sync_copy(acc_sh.at[pl.ds(base_row, ROWS_PER_TILE)],
                        out_hbm.at[c].at[pl.ds(base_row, ROWS_PER_TILE)])

    return k(x, row, col)


def _tc_combine(partials, W, b2d):
    BR = 2560

    def body(p_ref, w_ref, b_ref, o_ref):
        agg = p_ref[0] + p_ref[1]
        o_ref[...] = jnp.dot(agg, w_ref[...],
                             preferred_element_type=jnp.float32) + b_ref[...]

    return pl.pallas_call(
        body,
        grid=(ACC_ROWS // BR,),
        in_specs=[
            pl.BlockSpec((NC, BR, F), lambda i: (0, i, 0)),
            pl.BlockSpec((F, F), lambda i: (0, 0)),
            pl.BlockSpec((1, F), lambda i: (0, 0)),
        ],
        out_specs=pl.BlockSpec((BR, F), lambda i: (i, 0)),
        out_shape=jax.ShapeDtypeStruct((N_NODES, F), jnp.float32),
    )(partials, W, b2d)


def kernel(x, edge_index_or_adj, W, b):
    ei = edge_index_or_adj.astype(jnp.int32)
    row2d = ei[0].reshape(NC * NS, N_BATCH, EB)
    col = ei[1]
    partials = _sc_scatter(x, row2d, col)
    return _tc_combine(partials, W, b.reshape(1, F))


# speedup vs baseline: 2.2974x; 1.0153x over previous
import functools

import jax
import jax.numpy as jnp
from jax import lax
from jax.experimental import pallas as pl
from jax.experimental.pallas import tpu as pltpu
from jax.experimental.pallas import tpu_sc as plsc

N_NODES = 10000
N_EDGES = 320000
F = 128

NC = 2
NS = 16
EDGES_PER_CORE = N_EDGES // NC
EDGES_PER_TILE = EDGES_PER_CORE // NS
EB = 80
N_BATCH = EDGES_PER_TILE // EB
ACC_ROWS = 10240
ROWS_PER_TILE = ACC_ROWS // NS
ZR = 8


def _sc_scatter(x, row, col):
    mesh = plsc.VectorSubcoreMesh(
        core_axis_name="c", subcore_axis_name="s",
        num_cores=NC, num_subcores=NS)

    @functools.partial(
        pl.kernel,
        out_type=jax.ShapeDtypeStruct((NC, ACC_ROWS, F), jnp.float32),
        mesh=mesh,
        scratch_types=[
            pltpu.VMEM((N_BATCH * EB,), jnp.int32),
            pltpu.VMEM((N_BATCH * EB,), jnp.int32),
            pltpu.VMEM((EB, F), jnp.float32),
            pltpu.VMEM((EB, F), jnp.float32),
            pltpu.VMEM((ZR, F), jnp.float32),
            pltpu.VMEM_SHARED((ACC_ROWS, F), jnp.float32),
            pltpu.SemaphoreType.DMA,
            pltpu.SemaphoreType.DMA,
            pltpu.SemaphoreType.DMA,
        ],
        compiler_params=pltpu.CompilerParams(use_tc_tiling_on_sc=False),
    )
    def k(x_hbm, row_hbm, col_hbm, out_hbm, cidx_v, ridx_v, buf_a, buf_b,
          zbuf_v, acc_sh, sem_a, sem_b, sem_z):
        c = lax.axis_index("c")
        s = lax.axis_index("s")

        tid = c * NS + s
        idx_cp_c = pltpu.async_copy(
            col_hbm.at[pl.ds(tid * EDGES_PER_TILE, EDGES_PER_TILE)],
            cidx_v, sem_a)
        idx_cp_r = pltpu.async_copy(
            row_hbm.at[pl.ds(tid * EDGES_PER_TILE, EDGES_PER_TILE)],
            ridx_v, sem_b)

        zero16 = jnp.zeros((16,), jnp.float32)
        for i in range(ZR):
            for j in range(F // 16):
                zbuf_v[i, pl.ds(j * 16, 16)] = zero16

        base_row = s * ROWS_PER_TILE

        def zfire(i, carry):
            pltpu.async_copy(zbuf_v, acc_sh.at[pl.ds(base_row + i * ZR, ZR)],
                             sem_z)
            return carry

        def zdrain(i, carry):
            pltpu.make_async_copy(
                zbuf_v, acc_sh.at[pl.ds(base_row + i * ZR, ZR)],
                sem_z).wait()
            return carry

        def gather(i, buf, sem):
            pltpu.async_copy(x_hbm.at[cidx_v.at[pl.ds(i * EB, EB)]], buf, sem)

        def gwait(i, buf, sem):
            pltpu.make_async_copy(
                x_hbm.at[cidx_v.at[pl.ds(i * EB, EB)]], buf, sem).wait()

        def scat(i, buf):
            pltpu.sync_copy(buf, acc_sh.at[ridx_v.at[pl.ds(i * EB, EB)]],
                            add=True)

        lax.fori_loop(0, ROWS_PER_TILE // ZR, zfire, 0)
        idx_cp_c.wait()
        idx_cp_r.wait()
        gather(0, buf_a, sem_a)
        gather(1, buf_b, sem_b)
        lax.fori_loop(0, ROWS_PER_TILE // ZR, zdrain, 0)
        plsc.subcore_barrier()

        def eloop(j, carry):
            i = 2 * j
            gwait(i, buf_a, sem_a)
            scat(i, buf_a)
            gather(i + 2, buf_a, sem_a)
            gwait(i + 1, buf_b, sem_b)
            scat(i + 1, buf_b)
            gather(i + 3, buf_b, sem_b)
            return carry

        n_even = (N_BATCH // 2) * 2
        lax.fori_loop(0, n_even // 2 - 1, eloop, 0)
        gwait(n_even - 2, buf_a, sem_a)
        scat(n_even - 2, buf_a)
        gwait(n_even - 1, buf_b, sem_b)
        scat(n_even - 1, buf_b)
        for i in range(n_even, N_BATCH):
            gather(i, buf_a, sem_a)
            gwait(i, buf_a, sem_a)
            scat(i, buf_a)
        plsc.subcore_barrier()

        pltpu.sync_copy(acc_sh.at[pl.ds(base_row, ROWS_PER_TILE)],
                        out_hbm.at[c].at[pl.ds(base_row, ROWS_PER_TILE)])

    return k(x, row, col)


def _tc_combine(partials, W, b2d):
    BR = 2560

    def body(p_ref, w_ref, b_ref, o_ref):
        agg = p_ref[0] + p_ref[1]
        o_ref[...] = jnp.dot(agg, w_ref[...],
                             preferred_element_type=jnp.float32) + b_ref[...]

    return pl.pallas_call(
        body,
        grid=(ACC_ROWS // BR,),
        in_specs=[
            pl.BlockSpec((NC, BR, F), lambda i: (0, i, 0)),
            pl.BlockSpec((F, F), lambda i: (0, 0)),
            pl.BlockSpec((1, F), lambda i: (0, 0)),
        ],
        out_specs=pl.BlockSpec((BR, F), lambda i: (i, 0)),
        out_shape=jax.ShapeDtypeStruct((N_NODES, F), jnp.float32),
    )(partials, W, b2d)


def kernel(x, edge_index_or_adj, W, b):
    ei = edge_index_or_adj.astype(jnp.int32)
    partials = _sc_scatter(x, ei[0], ei[1])
    return _tc_combine(partials, W, b.reshape(1, F))


# speedup vs baseline: 2.4678x; 1.0742x over previous
import functools

import jax
import jax.numpy as jnp
from jax import lax
from jax.experimental import pallas as pl
from jax.experimental.pallas import tpu as pltpu
from jax.experimental.pallas import tpu_sc as plsc

N_NODES = 10000
N_EDGES = 320000
F = 128

NC = 2
NS = 16
EDGES_PER_CORE = N_EDGES // NC
EDGES_PER_TILE = EDGES_PER_CORE // NS
EB = 80
N_BATCH = EDGES_PER_TILE // EB
ACC_ROWS = 10240
ROWS_PER_TILE = ACC_ROWS // NS
ZR = 8


def _sc_scatter(x, ei):
    mesh = plsc.VectorSubcoreMesh(
        core_axis_name="c", subcore_axis_name="s",
        num_cores=NC, num_subcores=NS)

    @functools.partial(
        pl.kernel,
        out_type=jax.ShapeDtypeStruct((NC, ACC_ROWS, F), jnp.float32),
        mesh=mesh,
        scratch_types=[
            pltpu.VMEM((N_BATCH * EB,), jnp.int32),
            pltpu.VMEM((N_BATCH * EB,), jnp.int32),
            pltpu.VMEM((EB, F), jnp.float32),
            pltpu.VMEM((EB, F), jnp.float32),
            pltpu.VMEM((ZR, F), jnp.float32),
            pltpu.VMEM_SHARED((ACC_ROWS, F), jnp.float32),
            pltpu.SemaphoreType.DMA,
            pltpu.SemaphoreType.DMA,
            pltpu.SemaphoreType.DMA,
        ],
        compiler_params=pltpu.CompilerParams(use_tc_tiling_on_sc=False),
    )
    def k(x_hbm, ei_hbm, out_hbm, cidx_v, ridx_v, buf_a, buf_b,
          zbuf_v, acc_sh, sem_a, sem_b, sem_z):
        c = lax.axis_index("c")
        s = lax.axis_index("s")

        tid = c * NS + s
        idx_cp_c = pltpu.async_copy(
            ei_hbm.at[1, pl.ds(tid * EDGES_PER_TILE, EDGES_PER_TILE)],
            cidx_v, sem_a)
        idx_cp_r = pltpu.async_copy(
            ei_hbm.at[0, pl.ds(tid * EDGES_PER_TILE, EDGES_PER_TILE)],
            ridx_v, sem_b)

        zero16 = jnp.zeros((16,), jnp.float32)
        for i in range(ZR):
            for j in range(F // 16):
                zbuf_v[i, pl.ds(j * 16, 16)] = zero16

        base_row = s * ROWS_PER_TILE

        def zfire(i, carry):
            pltpu.async_copy(zbuf_v, acc_sh.at[pl.ds(base_row + i * ZR, ZR)],
                             sem_z)
            return carry

        def zdrain(i, carry):
            pltpu.make_async_copy(
                zbuf_v, acc_sh.at[pl.ds(base_row + i * ZR, ZR)],
                sem_z).wait()
            return carry

        def gather(i, buf, sem):
            pltpu.async_copy(x_hbm.at[cidx_v.at[pl.ds(i * EB, EB)]], buf, sem)

        def gwait(i, buf, sem):
            pltpu.make_async_copy(
                x_hbm.at[cidx_v.at[pl.ds(i * EB, EB)]], buf, sem).wait()

        def scat(i, buf):
            pltpu.sync_copy(buf, acc_sh.at[ridx_v.at[pl.ds(i * EB, EB)]],
                            add=True)

        lax.fori_loop(0, ROWS_PER_TILE // ZR, zfire, 0)
        idx_cp_c.wait()
        idx_cp_r.wait()
        gather(0, buf_a, sem_a)
        gather(1, buf_b, sem_b)
        lax.fori_loop(0, ROWS_PER_TILE // ZR, zdrain, 0)
        plsc.subcore_barrier()

        def eloop(j, carry):
            i = 2 * j
            gwait(i, buf_a, sem_a)
            scat(i, buf_a)
            gather(i + 2, buf_a, sem_a)
            gwait(i + 1, buf_b, sem_b)
            scat(i + 1, buf_b)
            gather(i + 3, buf_b, sem_b)
            return carry

        n_even = (N_BATCH // 2) * 2
        lax.fori_loop(0, n_even // 2 - 1, eloop, 0)
        gwait(n_even - 2, buf_a, sem_a)
        scat(n_even - 2, buf_a)
        gwait(n_even - 1, buf_b, sem_b)
        scat(n_even - 1, buf_b)
        for i in range(n_even, N_BATCH):
            gather(i, buf_a, sem_a)
            gwait(i, buf_a, sem_a)
            scat(i, buf_a)
        plsc.subcore_barrier()

        pltpu.sync_copy(acc_sh.at[pl.ds(base_row, ROWS_PER_TILE)],
                        out_hbm.at[c].at[pl.ds(base_row, ROWS_PER_TILE)])

    return k(x, ei)


def _tc_combine(partials, W, b2d):
    BR = 2560

    def body(p_ref, w_ref, b_ref, o_ref):
        agg = p_ref[0] + p_ref[1]
        o_ref[...] = jnp.dot(agg, w_ref[...],
                             preferred_element_type=jnp.float32) + b_ref[...]

    return pl.pallas_call(
        body,
        grid=(ACC_ROWS // BR,),
        in_specs=[
            pl.BlockSpec((NC, BR, F), lambda i: (0, i, 0)),
            pl.BlockSpec((F, F), lambda i: (0, 0)),
            pl.BlockSpec((1, F), lambda i: (0, 0)),
        ],
        out_specs=pl.BlockSpec((BR, F), lambda i: (i, 0)),
        out_shape=jax.ShapeDtypeStruct((N_NODES, F), jnp.float32),
    )(partials, W, b2d)


def kernel(x, edge_index_or_adj, W, b):
    ei = edge_index_or_adj.astype(jnp.int32)
    partials = _sc_scatter(x, ei)
    return _tc_combine(partials, W, b.reshape(1, F))
